# baseline (speedup 1.0000x reference)
import jax
import jax.numpy as jnp
import numpy as np
from jax import lax
from jax.experimental import pallas as pl
from jax.experimental.pallas import tpu as pltpu
from jax.experimental.pallas import tpu_sc as plsc

B, L, D = 4096, 200, 128
NC, NS, LANES = 2, 16, 16
NW = NC * NS
BPW = B // NW
NG = (L + LANES - 1) // LANES
LP = NG * LANES
HALF = 112
DK = D // LANES
SIGN_OFF = 0x7FFFFFFF
EPS_BITS = int(np.float32(1e-6).view(np.int32))
INF_BITS = 0x7F800000
NBUF = 4


def _row_masks(fb, g):
    mv = jnp.zeros((LANES,), jnp.int32)
    iota = lax.iota(jnp.int32, LANES)
    for tl in range(LANES):
        t = g * LANES + tl
        acc = jnp.zeros((LANES,), jnp.int32)
        for k in range(DK):
            v = fb[t, pl.ds(k * LANES, LANES)]
            vi = plsc.bitcast(v, jnp.int32) & SIGN_OFF
            acc = jnp.maximum(acc, vi)
        lanemask = acc <= EPS_BITS
        pc = plsc.all_reduce_population_count(lanemask)
        rowm = pc == LANES
        mv = mv | jnp.where(rowm & (iota == tl), 1, 0)
    return mv


def _body(x_hbm, out_hbm, s0, s1, s2, s3, fb, g_flat, g2, flags,
          ss0, ss1, ss2, ss3, cs0, cs1, cs2, cs3, gsem):
    wid = lax.axis_index("s") * NC + lax.axis_index("c")
    scrs = (s0, s1, s2, s3)
    sss = (ss0, ss1, ss2, ss3)
    css = (cs0, cs1, cs2, cs3)

    def scr_copy(j, k):
        base = (wid * BPW + j) * L
        return pltpu.make_async_copy(
            x_hbm.at[pl.ds(base, L), pl.ds(0, LANES)], scrs[k], sss[k])

    def hbm_copy(j, k):
        base = (wid * BPW + j) * L
        return pltpu.make_async_copy(
            x_hbm.at[pl.ds(base, L)], out_hbm.at[pl.ds(base, L)], css[k])

    for k in range(NBUF):
        flags[k] = 0
    scr_copy(0, 0).start()
    scr_copy(1, 1).start()

    def quad(i, _):
        for k in range(NBUF):
            j = NBUF * i + k
            b = wid * BPW + j
            base = b * L

            fl = flags[k]

            @pl.when((i >= 1) & (fl == 1))
            def _drain():
                hbm_copy(j, k).wait()

            kn = (k + 2) % NBUF
            if k < 2:
                scr_copy(j + 2, kn).start()
            else:
                @pl.when(i < BPW // NBUF - 1)
                def _nxt():
                    scr_copy(j + 2, kn).start()

            scr_copy(j, k).wait()
            scr = scrs[k]

            mn0 = jnp.full((LANES,), INF_BITS, jnp.int32)
            mn1 = jnp.full((LANES,), INF_BITS, jnp.int32)
            for t in range(L):
                v = scr[t, pl.ds(0, LANES)]
                vi = plsc.bitcast(v, jnp.int32) & SIGN_OFF
                if t % 2 == 0:
                    mn0 = jnp.minimum(mn0, vi)
                else:
                    mn1 = jnp.minimum(mn1, vi)
            mn = jnp.minimum(mn0, mn1)
            pc = plsc.all_reduce_population_count(mn <= EPS_BITS)
            ns = jnp.max(jnp.where(pc == LANES, 1, 0))

            flags[k] = 1

            @pl.when(ns == 0)
            def _clean():
                hbm_copy(j, k).start()

            @pl.when(ns > 0)
            def _full():
                pltpu.async_copy(
                    x_hbm.at[pl.ds(base, L)], fb.at[pl.ds(0, L)],
                    gsem).wait()

                def one_group(g, carry):
                    last_valid, nm_vec = carry
                    mv = _row_masks(fb, g)
                    t = g * LANES + lax.iota(jnp.int32, LANES)
                    oob = t > (L - 1)
                    masked = (mv == 1) & (~oob)
                    val = jnp.where(masked | oob, 0, t)
                    f_vec = jnp.maximum(
                        plsc.cummax(val),
                        jnp.full((LANES,), last_valid, jnp.int32))
                    nm_vec = nm_vec + plsc.all_reduce_population_count(masked)
                    g_flat[pl.ds(g * LANES, LANES)] = f_vec
                    return jnp.max(f_vec), nm_vec

                _, nm_vec = lax.fori_loop(
                    0, NG, one_group,
                    (jnp.int32(0), jnp.zeros((LANES,), jnp.int32)))
                nm_s = jnp.max(nm_vec)

                @pl.when(nm_s == 0)
                def _cleanish():
                    hbm_copy(j, k).start()

                @pl.when(nm_s > 0)
                def _rare():
                    base_vec = jnp.full((LANES,), base, jnp.int32)
                    for row in range(2):
                        for jj in range(HALF // LANES):
                            off = row * HALF + jj * LANES
                            if off < LP:
                                vec = g_flat[pl.ds(off, LANES)] + base_vec
                            else:
                                vec = base_vec
                            g2[row, pl.ds(jj * LANES, LANES)] = vec
                    pltpu.async_copy(
                        x_hbm.at[g2.at[0]], fb.at[pl.ds(0, HALF)],
                        gsem).wait()
                    pltpu.async_copy(
                        x_hbm.at[g2.at[1]], fb.at[pl.ds(HALF, HALF)],
                        gsem).wait()
                    pltpu.async_copy(
                        fb.at[pl.ds(0, L)], out_hbm.at[pl.ds(base, L)],
                        gsem).wait()
                    flags[k] = 0
        return 0

    lax.fori_loop(0, BPW // NBUF, quad, 0)
    for k in range(NBUF):
        fl = flags[k]

        @pl.when(fl == 1)
        def _final_drain():
            hbm_copy(BPW - NBUF + k, k).wait()


@jax.jit
def _imputer(xf):
    mesh = plsc.VectorSubcoreMesh(core_axis_name="c", subcore_axis_name="s")
    return pl.kernel(
        _body,
        out_type=jax.ShapeDtypeStruct((B * L, D), jnp.float32),
        mesh=mesh,
        compiler_params=pltpu.CompilerParams(
            needs_layout_passes=False, use_tc_tiling_on_sc=False),
        scratch_types=(
            [pltpu.VMEM((L, LANES), jnp.float32) for _ in range(NBUF)]
            + [pltpu.VMEM((2 * HALF, D), jnp.float32),
               pltpu.VMEM((LP,), jnp.int32),
               pltpu.VMEM((2, HALF), jnp.int32),
               pltpu.SMEM((NBUF,), jnp.int32)]
            + [pltpu.SemaphoreType.DMA for _ in range(2 * NBUF + 1)]
        ),
    )(xf)


def kernel(x):
    batch_dims = x.shape[:-2]
    xf = x.reshape(B * L, D)
    return _imputer(xf).reshape(*batch_dims, L, D)
